# Initial kernel scaffold; baseline (speedup 1.0000x reference)
import jax
import jax.numpy as jnp
from jax import lax
from jax.experimental import pallas as pl
from jax.experimental.pallas import tpu as pltpu
from jax.experimental.pallas import tpu_sc as plsc

NF = 8
D = 64
V = 35
DW = NF * D
CH = 64
NWORK = 32


def _body(xoff_hbm, wcat_hbm, out_hbm, tbl_v, idx_v, buf0, buf1, sem0, sem1):
    L = idx_v.shape[1]
    nch = L // CH
    wid = lax.axis_index("s") * 2 + lax.axis_index("c")

    pltpu.sync_copy(wcat_hbm, tbl_v)
    pltpu.sync_copy(xoff_hbm.at[wid], idx_v)

    bufs = (buf0, buf1)
    sems = (sem0, sem1)

    def fill(c, buf):
        def tok(t, carry):
            dst_t = t * DW
            for i in range(NF):
                v = idx_v[i, c * CH + t]
                src = v * D
                for k in range(D // 16):
                    buf[pl.ds(dst_t + i * D + k * 16, 16)] = (
                        tbl_v[pl.ds(src + k * 16, 16)])
            return carry
        lax.fori_loop(0, CH, tok, 0)

    inflight = [None, None]
    for c in range(nch):
        slot = c % 2
        if inflight[slot] is not None:
            inflight[slot].wait()
        fill(c, bufs[slot])
        inflight[slot] = pltpu.async_copy(
            bufs[slot], out_hbm.at[wid, pl.ds(c * CH * DW, CH * DW)],
            sems[slot])
    inflight[0].wait()
    inflight[1].wait()


def kernel(x, W0, W1, W2, W3, W4, W5, W6, W7):
    B, nf, L = x.shape
    assert nf == NF and B == NWORK and L % CH == 0
    tables = (W0, W1, W2, W3, W4, W5, W6, W7)
    wcat = jnp.concatenate([w[:V] for w in tables], axis=0).reshape(-1)
    xoff = x.astype(jnp.int32) + (V * jnp.arange(NF, dtype=jnp.int32))[None, :, None]

    mesh = plsc.VectorSubcoreMesh(core_axis_name="c", subcore_axis_name="s")
    f = pl.kernel(
        _body,
        out_type=jax.ShapeDtypeStruct((B, L * DW), jnp.float32),
        mesh=mesh,
        scratch_types=[
            pltpu.VMEM((NF * V * D,), jnp.float32),
            pltpu.VMEM((NF, L), jnp.int32),
            pltpu.VMEM((CH * DW,), jnp.float32),
            pltpu.VMEM((CH * DW,), jnp.float32),
            pltpu.SemaphoreType.DMA,
            pltpu.SemaphoreType.DMA,
        ],
    )
    out = f(xoff, wcat)
    return out.reshape(B, L, DW)


# speedup vs baseline: 1.5519x; 1.5519x over previous
import jax
import jax.numpy as jnp
from jax import lax
from jax.experimental import pallas as pl
from jax.experimental.pallas import tpu as pltpu
from jax.experimental.pallas import tpu_sc as plsc

NF = 8
D = 64
V = 35
DW = NF * D
CH = 64
CHW = CH * DW
NWORK = 32


def _body(xoff_hbm, wcat_hbm, out_hbm, tbl_v, idx_v, buf0, buf1, sem0, sem1):
    L = idx_v.shape[1]
    nch = L // CH
    wid = lax.axis_index("s") * 2 + lax.axis_index("c")

    pltpu.sync_copy(wcat_hbm, tbl_v)
    pltpu.sync_copy(xoff_hbm.at[wid], idx_v)

    lane = lax.iota(jnp.int32, 16)
    dst_lane = lane * DW

    def fill(c, buf):
        def group(g, _):
            def field(i, _):
                idxv = idx_v[i, pl.ds(c * CH + g * 16, 16)]
                src0 = idxv * D
                dst0 = dst_lane + (g * (16 * DW) + i * D)
                def col(j, _):
                    vals = plsc.load_gather(tbl_v, [src0 + j])
                    plsc.store_scatter(buf, [dst0 + j], vals)
                    return 0
                lax.fori_loop(0, D, col, 0, unroll=8)
                return 0
            lax.fori_loop(0, NF, field, 0)
            return 0
        lax.fori_loop(0, CH // 16, group, 0)

    bufs = (buf0, buf1)
    sems = (sem0, sem1)

    for c in range(2):
        fill(c, bufs[c])
        pltpu.async_copy(bufs[c], out_hbm.at[wid, pl.ds(c * CHW, CHW)], sems[c])

    def pair(o, _):
        for phase in range(2):
            c = 2 * o + phase
            pltpu.make_async_copy(
                bufs[phase], out_hbm.at[wid, pl.ds(0, CHW)], sems[phase]).wait()
            fill(c, bufs[phase])
            pltpu.async_copy(
                bufs[phase], out_hbm.at[wid, pl.ds(c * CHW, CHW)], sems[phase])
        return 0
    lax.fori_loop(1, nch // 2, pair, 0)

    for phase in range(2):
        pltpu.make_async_copy(
            bufs[phase], out_hbm.at[wid, pl.ds(0, CHW)], sems[phase]).wait()


def kernel(x, W0, W1, W2, W3, W4, W5, W6, W7):
    B, nf, L = x.shape
    assert nf == NF and B == NWORK and L % (2 * CH) == 0
    tables = (W0, W1, W2, W3, W4, W5, W6, W7)
    wcat = jnp.concatenate([w[:V] for w in tables], axis=0).reshape(-1)
    xoff = x.astype(jnp.int32) + (V * jnp.arange(NF, dtype=jnp.int32))[None, :, None]

    mesh = plsc.VectorSubcoreMesh(core_axis_name="c", subcore_axis_name="s")
    f = pl.kernel(
        _body,
        compiler_params=pltpu.CompilerParams(
            use_tc_tiling_on_sc=False, needs_layout_passes=False),
        out_type=jax.ShapeDtypeStruct((B, L * DW), jnp.float32),
        mesh=mesh,
        scratch_types=[
            pltpu.VMEM((NF * V * D,), jnp.float32),
            pltpu.VMEM((NF, L), jnp.int32),
            pltpu.VMEM((CHW,), jnp.float32),
            pltpu.VMEM((CHW,), jnp.float32),
            pltpu.SemaphoreType.DMA,
            pltpu.SemaphoreType.DMA,
        ],
    )
    out = f(xoff, wcat)
    return out.reshape(B, L, DW)


# speedup vs baseline: 5.1733x; 3.3336x over previous
import jax
import jax.numpy as jnp
from jax import lax
from jax.experimental import pallas as pl
from jax.experimental.pallas import tpu as pltpu
from jax.experimental.pallas import tpu_sc as plsc

NF = 8
D = 64
V = 35
DW = NF * D
CH = 64
CHW = CH * DW
NWORK = 32


def _body(xoff_hbm, wcat_hbm, out_hbm, tbl_v, idx_v, buf0, buf1, sem0, sem1):
    L = idx_v.shape[1]
    nch = L // CH
    wid = lax.axis_index("s") * 2 + lax.axis_index("c")

    pltpu.sync_copy(wcat_hbm, tbl_v)
    pltpu.sync_copy(xoff_hbm.at[wid], idx_v)

    def fill(c, buf):
        def group(g, _):
            base = c * CH + g * 16
            def field(i, _):
                idxv = idx_v[i, pl.ds(base, 16)]
                dst_gi = g * (16 * DW) + i * D
                for t in range(16):
                    src = idxv[t] * D
                    dst = dst_gi + t * DW
                    for k in range(D // 16):
                        buf[pl.ds(dst + k * 16, 16)] = (
                            tbl_v[pl.ds(src + k * 16, 16)])
                return 0
            lax.fori_loop(0, NF, field, 0)
            return 0
        lax.fori_loop(0, CH // 16, group, 0)

    bufs = (buf0, buf1)
    sems = (sem0, sem1)

    for c in range(2):
        fill(c, bufs[c])
        pltpu.async_copy(bufs[c], out_hbm.at[wid, pl.ds(c * CHW, CHW)], sems[c])

    def pair(o, _):
        for phase in range(2):
            c = 2 * o + phase
            pltpu.make_async_copy(
                bufs[phase], out_hbm.at[wid, pl.ds(0, CHW)], sems[phase]).wait()
            fill(c, bufs[phase])
            pltpu.async_copy(
                bufs[phase], out_hbm.at[wid, pl.ds(c * CHW, CHW)], sems[phase])
        return 0
    lax.fori_loop(1, nch // 2, pair, 0)

    for phase in range(2):
        pltpu.make_async_copy(
            bufs[phase], out_hbm.at[wid, pl.ds(0, CHW)], sems[phase]).wait()


def kernel(x, W0, W1, W2, W3, W4, W5, W6, W7):
    B, nf, L = x.shape
    assert nf == NF and B == NWORK and L % (2 * CH) == 0
    tables = (W0, W1, W2, W3, W4, W5, W6, W7)
    wcat = jnp.concatenate([w[:V] for w in tables], axis=0).reshape(-1)
    xoff = x.astype(jnp.int32) + (V * jnp.arange(NF, dtype=jnp.int32))[None, :, None]

    mesh = plsc.VectorSubcoreMesh(core_axis_name="c", subcore_axis_name="s")
    f = pl.kernel(
        _body,
        compiler_params=pltpu.CompilerParams(
            use_tc_tiling_on_sc=False, needs_layout_passes=False),
        out_type=jax.ShapeDtypeStruct((B, L * DW), jnp.float32),
        mesh=mesh,
        scratch_types=[
            pltpu.VMEM((NF * V * D,), jnp.float32),
            pltpu.VMEM((NF, L), jnp.int32),
            pltpu.VMEM((CHW,), jnp.float32),
            pltpu.VMEM((CHW,), jnp.float32),
            pltpu.SemaphoreType.DMA,
            pltpu.SemaphoreType.DMA,
        ],
    )
    out = f(xoff, wcat)
    return out.reshape(B, L, DW)
